# baseline (speedup 1.0000x reference)
import numpy as np
import jax
import jax.numpy as jnp
from jax import lax
from jax.experimental import pallas as pl
from jax.experimental.pallas import tpu as pltpu

_TEMP = 0.5
_EPS = 1e-10
_KS = (np.uint32(0), np.uint32(42), np.uint32(0x1BD11BF0))
_ROT = ((13, 15, 26, 6), (17, 29, 16, 24))
_TB = 128


def _threefry_bits(lo):
    x0 = jnp.zeros_like(lo) + _KS[0]
    x1 = lo + _KS[1]
    for i in range(5):
        for r in _ROT[i % 2]:
            x0 = x0 + x1
            x1 = (x1 << r) | (x1 >> (32 - r))
            x1 = x0 ^ x1
        x0 = x0 + _KS[(i + 1) % 3]
        x1 = x1 + _KS[(i + 2) % 3] + np.uint32(i + 1)
    return x0 ^ x1


def _vq_kernel(prec_ref, z_ref, cbt_ref, cb_ref, zq_ref, loss_ref, perp_ref,
               csq_ref, avg_ref, acc_ref):
    i = pl.program_id(0)
    nb = pl.num_programs(0)
    tb, dim = z_ref.shape
    k = cb_ref.shape[0]
    bs = nb * tb

    prec = prec_ref[0]
    s = -(0.5 * prec)

    @pl.when(i == 0)
    def _init():
        cb = cb_ref[...]
        csq_col = jnp.sum(cb * cb, axis=1, keepdims=True)
        csq_ref[...] = csq_col.reshape(1, k)
        avg_ref[...] = jnp.zeros_like(avg_ref)
        acc_ref[0] = 0.0
        acc_ref[1] = 0.0

    z = z_ref[...]
    zsq = jnp.sum(z * z, axis=1, keepdims=True)
    zc = lax.dot_general(z, cbt_ref[...], (((1,), (0,)), ((), ())),
                         preferred_element_type=jnp.float32,
                         precision=lax.Precision.DEFAULT)
    distances = (zsq + csq_ref[...]) - 2.0 * zc
    logit = s * distances

    m = jnp.max(logit, axis=1, keepdims=True)
    shifted = logit - m
    p_un = jnp.exp(shifted)
    zden = jnp.sum(p_un, axis=1, keepdims=True)
    p = p_un / zden
    avg_ref[...] += jnp.sum(p, axis=0, keepdims=True)
    logz = jnp.log(zden)
    acc_ref[0] += jnp.sum(p * (shifted - logz))

    base = np.uint32(0) + (i * tb * k)
    lo = (base.astype(jnp.uint32)
          + lax.broadcasted_iota(jnp.uint32, (tb, k), 0) * np.uint32(k)
          + lax.broadcasted_iota(jnp.uint32, (tb, k), 1))
    bits = _threefry_bits(lo)
    u = lax.bitcast_convert_type(
        (bits >> np.uint32(9)) | np.uint32(0x3F800000), jnp.float32) - 1.0
    g = -jnp.log(-jnp.log(u + _EPS) + _EPS)

    el = (logit + g) / _TEMP
    m2 = jnp.max(el, axis=1, keepdims=True)
    e_un = jnp.exp(el - m2)
    z2 = jnp.sum(e_un, axis=1, keepdims=True)
    enc = e_un / z2
    zq = lax.dot_general(enc, cb_ref[...], (((1,), (0,)), ((), ())),
                         preferred_element_type=jnp.float32,
                         precision=lax.Precision.HIGHEST)
    zq_ref[...] = zq
    diff = z - zq
    acc_ref[1] += jnp.sum(diff * diff)

    @pl.when(i == nb - 1)
    def _fin():
        avg = avg_ref[...] / bs
        h = jnp.sum(avg * jnp.log(avg + 1e-7))
        perp_ref[0] = jnp.exp(-h)
        kld_d = acc_ref[0] / bs
        kld_c = (0.5 * prec) * acc_ref[1] / bs
        loss_ref[0] = kld_d + kld_c


def kernel(z_from_encoder, param_q, codebook, flg_train=True):
    bs, dim = z_from_encoder.shape
    k = codebook.shape[0]
    tb = _TB if bs % _TB == 0 else bs
    nb = bs // tb
    precision = 1.0 / jnp.clip(param_q, 1e-10, None)
    cbt = codebook.T

    zq, loss, perp = pl.pallas_call(
        _vq_kernel,
        grid=(nb,),
        in_specs=[
            pl.BlockSpec(memory_space=pltpu.SMEM),
            pl.BlockSpec((tb, dim), lambda i: (i, 0)),
            pl.BlockSpec((dim, k), lambda i: (0, 0)),
            pl.BlockSpec((k, dim), lambda i: (0, 0)),
        ],
        out_specs=[
            pl.BlockSpec((tb, dim), lambda i: (i, 0)),
            pl.BlockSpec(memory_space=pltpu.SMEM),
            pl.BlockSpec(memory_space=pltpu.SMEM),
        ],
        out_shape=[
            jax.ShapeDtypeStruct((bs, dim), jnp.float32),
            jax.ShapeDtypeStruct((1,), jnp.float32),
            jax.ShapeDtypeStruct((1,), jnp.float32),
        ],
        scratch_shapes=[
            pltpu.VMEM((1, k), jnp.float32),
            pltpu.VMEM((1, k), jnp.float32),
            pltpu.SMEM((2,), jnp.float32),
        ],
        compiler_params=pltpu.CompilerParams(
            dimension_semantics=("arbitrary",)),
    )(precision, z_from_encoder, cbt, codebook)
    return zq, loss[0], perp[0]


# speedup vs baseline: 1.1732x; 1.1732x over previous
import numpy as np
import jax
import jax.numpy as jnp
from jax import lax
from jax.experimental import pallas as pl
from jax.experimental.pallas import tpu as pltpu

_TEMP = 0.5
_EPS = 1e-10
_KS = (np.uint32(0), np.uint32(42), np.uint32(0x1BD11BF0))
_ROT = ((13, 15, 26, 6), (17, 29, 16, 24))
_TB = 128


def _threefry_bits(lo):
    x0 = jnp.zeros_like(lo) + _KS[0]
    x1 = lo + _KS[1]
    for i in range(5):
        for r in _ROT[i % 2]:
            x0 = x0 + x1
            x1 = (x1 << r) | (x1 >> (32 - r))
            x1 = x0 ^ x1
        x0 = x0 + _KS[(i + 1) % 3]
        x1 = x1 + _KS[(i + 2) % 3] + np.uint32(i + 1)
    return x0 ^ x1


def _vq_kernel(prec_ref, z_ref, cbt_ref, cb_ref, zq_ref, loss_ref, perp_ref,
               csq_ref, avg_ref, acc_ref, lo0_ref, cbtb_ref, cbb_ref):
    i = pl.program_id(0)
    nb = pl.num_programs(0)
    tb, dim = z_ref.shape
    k = cb_ref.shape[0]
    bs = nb * tb

    prec = prec_ref[0]
    s = -(0.5 * prec)

    @pl.when(i == 0)
    def _init():
        cbt = cbt_ref[...]
        csq_ref[...] = jnp.sum(cbt * cbt, axis=0, keepdims=True)
        avg_ref[...] = jnp.zeros_like(avg_ref)
        acc_ref[0] = 0.0
        acc_ref[1] = 0.0
        lo0_ref[...] = (lax.broadcasted_iota(jnp.uint32, (tb, k), 0)
                        * np.uint32(k)
                        + lax.broadcasted_iota(jnp.uint32, (tb, k), 1))
        cbtb_ref[...] = cbt.astype(jnp.bfloat16)
        cbb_ref[...] = cb_ref[...].astype(jnp.bfloat16)

    z = z_ref[...]
    zsq = jnp.sum(z * z, axis=1, keepdims=True)
    zc = lax.dot_general(z.astype(jnp.bfloat16), cbtb_ref[...],
                         (((1,), (0,)), ((), ())),
                         preferred_element_type=jnp.float32,
                         precision=lax.Precision.DEFAULT)
    distances = (zsq + csq_ref[...]) - 2.0 * zc
    logit = s * distances

    m = jnp.max(logit, axis=1, keepdims=True)
    shifted = logit - m
    p_un = jnp.exp(shifted)
    zden = jnp.sum(p_un, axis=1, keepdims=True)
    p = p_un / zden
    avg_ref[...] += jnp.sum(p, axis=0, keepdims=True)
    logz = jnp.log(zden)
    acc_ref[0] += jnp.sum(p * (shifted - logz))

    base = (i * tb * k).astype(jnp.uint32)
    lo = lo0_ref[...] + base
    bits = _threefry_bits(lo)
    u = lax.bitcast_convert_type(
        (bits >> np.uint32(9)) | np.uint32(0x3F800000), jnp.float32) - 1.0
    g = -jnp.log(-jnp.log(u + _EPS) + _EPS)

    el = (logit + g) / _TEMP
    m2 = jnp.max(el, axis=1, keepdims=True)
    e_un = jnp.exp(el - m2)
    z2 = jnp.sum(e_un, axis=1, keepdims=True)
    enc = e_un / z2
    zq = lax.dot_general(enc.astype(jnp.bfloat16), cbb_ref[...],
                         (((1,), (0,)), ((), ())),
                         preferred_element_type=jnp.float32,
                         precision=lax.Precision.DEFAULT)
    zq_ref[...] = zq
    diff = z - zq
    acc_ref[1] += jnp.sum(diff * diff)

    @pl.when(i == nb - 1)
    def _fin():
        avg = avg_ref[...] / bs
        h = jnp.sum(avg * jnp.log(avg + 1e-7))
        perp_ref[0] = jnp.exp(-h)
        kld_d = acc_ref[0] / bs
        kld_c = (0.5 * prec) * acc_ref[1] / bs
        loss_ref[0] = kld_d + kld_c


def kernel(z_from_encoder, param_q, codebook, flg_train=True):
    bs, dim = z_from_encoder.shape
    k = codebook.shape[0]
    tb = _TB if bs % _TB == 0 else bs
    nb = bs // tb
    precision = 1.0 / jnp.clip(param_q, 1e-10, None)
    cbt = codebook.T

    zq, loss, perp = pl.pallas_call(
        _vq_kernel,
        grid=(nb,),
        in_specs=[
            pl.BlockSpec(memory_space=pltpu.SMEM),
            pl.BlockSpec((tb, dim), lambda i: (i, 0)),
            pl.BlockSpec((dim, k), lambda i: (0, 0)),
            pl.BlockSpec((k, dim), lambda i: (0, 0)),
        ],
        out_specs=[
            pl.BlockSpec((tb, dim), lambda i: (i, 0)),
            pl.BlockSpec(memory_space=pltpu.SMEM),
            pl.BlockSpec(memory_space=pltpu.SMEM),
        ],
        out_shape=[
            jax.ShapeDtypeStruct((bs, dim), jnp.float32),
            jax.ShapeDtypeStruct((1,), jnp.float32),
            jax.ShapeDtypeStruct((1,), jnp.float32),
        ],
        scratch_shapes=[
            pltpu.VMEM((1, k), jnp.float32),
            pltpu.VMEM((1, k), jnp.float32),
            pltpu.SMEM((2,), jnp.float32),
            pltpu.VMEM((tb, k), jnp.uint32),
            pltpu.VMEM((dim, k), jnp.bfloat16),
            pltpu.VMEM((k, dim), jnp.bfloat16),
        ],
        compiler_params=pltpu.CompilerParams(
            dimension_semantics=("arbitrary",)),
    )(precision, z_from_encoder, cbt, codebook)
    return zq, loss[0], perp[0]


# speedup vs baseline: 1.2107x; 1.0320x over previous
import numpy as np
import jax
import jax.numpy as jnp
from jax import lax
from jax.experimental import pallas as pl
from jax.experimental.pallas import tpu as pltpu

_TEMP = 0.5
_EPS = 1e-10
_KS = (np.uint32(0), np.uint32(42), np.uint32(0x1BD11BF0))
_ROT = ((13, 15, 26, 6), (17, 29, 16, 24))
_TB = 128


def _threefry_bits(lo):
    x1 = lo + _KS[1]
    x0 = x1
    x1 = (x1 << 13) | (x1 >> 19)
    x1 = x0 ^ x1
    for r in (15, 26, 6):
        x0 = x0 + x1
        x1 = (x1 << r) | (x1 >> (32 - r))
        x1 = x0 ^ x1
    x0 = x0 + _KS[1]
    x1 = x1 + np.uint32((int(_KS[2]) + 1) & 0xFFFFFFFF)
    for i in range(1, 5):
        for r in _ROT[i % 2]:
            x0 = x0 + x1
            x1 = (x1 << r) | (x1 >> (32 - r))
            x1 = x0 ^ x1
        x0 = x0 + _KS[(i + 1) % 3]
        x1 = x1 + np.uint32((int(_KS[(i + 2) % 3]) + i + 1) & 0xFFFFFFFF)
    return x0 ^ x1


def _vq_kernel(prec_ref, z_ref, cbt_ref, cb_ref, zq_ref, loss_ref, perp_ref,
               csq_ref, avg_ref, acc_ref, lo0_ref, cbtb_ref, cbb_ref):
    i = pl.program_id(0)
    nb = pl.num_programs(0)
    tb, dim = z_ref.shape
    k = cb_ref.shape[0]
    bs = nb * tb

    prec = prec_ref[0]
    s = -(0.5 * prec)

    @pl.when(i == 0)
    def _init():
        cbt = cbt_ref[...]
        csq_ref[...] = jnp.sum(cbt * cbt, axis=0, keepdims=True)
        avg_ref[...] = jnp.zeros_like(avg_ref)
        acc_ref[0] = 0.0
        acc_ref[1] = 0.0
        lo0_ref[...] = (lax.broadcasted_iota(jnp.uint32, (tb, k), 0)
                        * np.uint32(k)
                        + lax.broadcasted_iota(jnp.uint32, (tb, k), 1))
        cbtb_ref[...] = cbt.astype(jnp.bfloat16)
        cbb_ref[...] = cb_ref[...].astype(jnp.bfloat16)

    z = z_ref[...]
    zsq = jnp.sum(z * z, axis=1, keepdims=True)
    zc = lax.dot_general(z.astype(jnp.bfloat16), cbtb_ref[...],
                         (((1,), (0,)), ((), ())),
                         preferred_element_type=jnp.float32,
                         precision=lax.Precision.DEFAULT)
    distances = (zsq + csq_ref[...]) - 2.0 * zc
    logit = s * distances

    m = jnp.max(logit, axis=1, keepdims=True)
    shifted = logit - m
    p_un = jnp.exp(shifted)
    zden = jnp.sum(p_un, axis=1, keepdims=True)
    p = p_un * (1.0 / zden)
    avg_ref[...] += jnp.sum(p, axis=0, keepdims=True)
    logz = jnp.log(zden)
    acc_ref[0] += jnp.sum(p * shifted) - jnp.sum(logz)

    base = (i * tb * k).astype(jnp.uint32)
    lo = lo0_ref[...] + base
    bits = _threefry_bits(lo)
    u = lax.bitcast_convert_type(
        (bits >> np.uint32(9)) | np.uint32(0x3F800000), jnp.float32) - 1.0
    gl = jnp.log(_EPS - jnp.log(u + _EPS))

    t = logit - gl
    mt = jnp.max(t, axis=1, keepdims=True)
    e_un = jnp.exp((t - mt) * 2.0)
    z2 = jnp.sum(e_un, axis=1, keepdims=True)
    enc = e_un * (1.0 / z2)
    zq = lax.dot_general(enc.astype(jnp.bfloat16), cbb_ref[...],
                         (((1,), (0,)), ((), ())),
                         preferred_element_type=jnp.float32,
                         precision=lax.Precision.DEFAULT)
    zq_ref[...] = zq
    diff = z - zq
    acc_ref[1] += jnp.sum(diff * diff)

    @pl.when(i == nb - 1)
    def _fin():
        avg = avg_ref[...] / bs
        h = jnp.sum(avg * jnp.log(avg + 1e-7))
        perp_ref[0] = jnp.exp(-h)
        kld_d = acc_ref[0] / bs
        kld_c = (0.5 * prec) * acc_ref[1] / bs
        loss_ref[0] = kld_d + kld_c


def kernel(z_from_encoder, param_q, codebook, flg_train=True):
    bs, dim = z_from_encoder.shape
    k = codebook.shape[0]
    tb = _TB if bs % _TB == 0 else bs
    nb = bs // tb
    precision = 1.0 / jnp.clip(param_q, 1e-10, None)
    cbt = codebook.T

    zq, loss, perp = pl.pallas_call(
        _vq_kernel,
        grid=(nb,),
        in_specs=[
            pl.BlockSpec(memory_space=pltpu.SMEM),
            pl.BlockSpec((tb, dim), lambda i: (i, 0)),
            pl.BlockSpec((dim, k), lambda i: (0, 0)),
            pl.BlockSpec((k, dim), lambda i: (0, 0)),
        ],
        out_specs=[
            pl.BlockSpec((tb, dim), lambda i: (i, 0)),
            pl.BlockSpec(memory_space=pltpu.SMEM),
            pl.BlockSpec(memory_space=pltpu.SMEM),
        ],
        out_shape=[
            jax.ShapeDtypeStruct((bs, dim), jnp.float32),
            jax.ShapeDtypeStruct((1,), jnp.float32),
            jax.ShapeDtypeStruct((1,), jnp.float32),
        ],
        scratch_shapes=[
            pltpu.VMEM((1, k), jnp.float32),
            pltpu.VMEM((1, k), jnp.float32),
            pltpu.SMEM((2,), jnp.float32),
            pltpu.VMEM((tb, k), jnp.uint32),
            pltpu.VMEM((dim, k), jnp.bfloat16),
            pltpu.VMEM((k, dim), jnp.bfloat16),
        ],
        compiler_params=pltpu.CompilerParams(
            dimension_semantics=("arbitrary",)),
    )(precision, z_from_encoder, cbt, codebook)
    return zq, loss[0], perp[0]
